# Initial kernel scaffold; baseline (speedup 1.0000x reference)
import jax
import jax.numpy as jnp
from jax.experimental import pallas as pl


def kernel(x, edge_index, edge_attr, nn1_w1, nn1_b1, nn1_w2, nn1_b2, root1_w, bias1, nn2_w1, nn2_b1, nn2_w2, nn2_b2, root2_w, bias2, cls_w1, cls_b1, cls_w2, cls_b2):
    raise NotImplementedError("write your pallas kernel here")


# speedup vs baseline: 2.4989x; 2.4989x over previous
import functools

import jax
import jax.numpy as jnp
from jax import lax
from jax.experimental import pallas as pl
from jax.experimental.pallas import tpu as pltpu
from jax.experimental.pallas import tpu_sc as plsc

_NW = 32
_CH = 128


def _sc_gather(table, idx):
    e_pad = idx.shape[0]
    d = table.shape[1]
    b_per_w = e_pad // _NW
    n_ch = b_per_w // _CH
    mesh = plsc.VectorSubcoreMesh(core_axis_name="c", subcore_axis_name="s")

    @functools.partial(
        pl.kernel, mesh=mesh,
        out_type=jax.ShapeDtypeStruct((e_pad, d), jnp.float32),
        compiler_params=pltpu.CompilerParams(use_tc_tiling_on_sc=False),
        scratch_types=[
            pltpu.VMEM((_CH,), jnp.int32),
            pltpu.VMEM((_CH, d), jnp.float32),
            pltpu.SemaphoreType.DMA,
        ],
    )
    def k(table_hbm, idx_hbm, out_hbm, idx_v, rows_v, sem):
        wid = lax.axis_index("s") * 2 + lax.axis_index("c")
        base = wid * b_per_w

        def body(c, carry):
            off = base + c * _CH
            pltpu.sync_copy(idx_hbm.at[pl.ds(off, _CH)], idx_v)
            pltpu.async_copy(table_hbm.at[idx_v], rows_v, sem).wait()
            pltpu.sync_copy(rows_v, out_hbm.at[pl.ds(off, _CH)])
            return carry

        lax.fori_loop(0, n_ch, body, 0)

    return k(table, idx)


def _sc_scatter_add(vals, idx, zeros, n_acc):
    e_pad = idx.shape[0]
    b_per_w = e_pad // _NW
    n_ch = b_per_w // _CH
    n_slice = n_acc // 16
    mesh = plsc.VectorSubcoreMesh(core_axis_name="c", subcore_axis_name="s")

    @functools.partial(
        pl.kernel, mesh=mesh,
        out_type=jax.ShapeDtypeStruct((2, n_acc, 16), jnp.float32),
        compiler_params=pltpu.CompilerParams(use_tc_tiling_on_sc=False),
        scratch_types=[
            pltpu.VMEM((_CH,), jnp.int32),
            pltpu.VMEM((_CH, 16), jnp.float32),
            pltpu.VMEM_SHARED((n_acc, 16), jnp.float32),
        ],
    )
    def k(vals_hbm, idx_hbm, zeros_hbm, out_hbm, idx_v, rows_v, acc_sh):
        cid = lax.axis_index("c")
        sid = lax.axis_index("s")
        wid = sid * 2 + cid
        base = wid * b_per_w
        pltpu.sync_copy(zeros_hbm.at[pl.ds(sid * n_slice, n_slice)],
                        acc_sh.at[pl.ds(sid * n_slice, n_slice)])
        plsc.subcore_barrier()

        def body(c, carry):
            off = base + c * _CH
            pltpu.sync_copy(idx_hbm.at[pl.ds(off, _CH)], idx_v)
            pltpu.sync_copy(vals_hbm.at[pl.ds(off, _CH)], rows_v)
            pltpu.sync_copy(rows_v, acc_sh.at[idx_v], add=True)
            return carry

        lax.fori_loop(0, n_ch, body, 0)
        plsc.subcore_barrier()
        pltpu.sync_copy(acc_sh.at[pl.ds(sid * n_slice, n_slice)],
                        out_hbm.at[cid, pl.ds(sid * n_slice, n_slice)])

    return k(vals, idx, zeros)


def _msg1_body(ea_ref, xj_ref, w1_ref, b1_ref, w2r_ref, b2r_ref, out_ref):
    b = ea_ref.shape[0]
    h = jnp.maximum(
        jnp.dot(ea_ref[...], w1_ref[...], preferred_element_type=jnp.float32)
        + b1_ref[...], 0.0)
    xj = xj_ref[...]
    col = lax.broadcasted_iota(jnp.int32, (b, 16), 1)
    m = jnp.where(col == 4, 1.0, 0.0)
    for o in range(4):
        c = jnp.dot(h, w2r_ref[o], preferred_element_type=jnp.float32)
        c = c + b2r_ref[o:o + 1, :]
        rs = jnp.sum(xj * c, axis=1, keepdims=True)
        m = m + rs * jnp.where(col == o, 1.0, 0.0)
    out_ref[...] = m


def _msg2_body(ea_ref, hj_ref, w1_ref, b1_ref, w2_ref, b2_ref, out_ref):
    h2e = jnp.maximum(
        jnp.dot(ea_ref[...], w1_ref[...], preferred_element_type=jnp.float32)
        + b1_ref[...], 0.0)
    ew2 = jnp.dot(h2e, w2_ref[...], preferred_element_type=jnp.float32) \
        + b2_ref[...]
    ri = lax.broadcasted_iota(jnp.int32, (16, 16), 0)
    rj = lax.broadcasted_iota(jnp.int32, (16, 16), 1)
    r2 = jnp.where(ri == rj // 4, 1.0, 0.0)
    s2 = jnp.where(rj == ri % 4, 1.0, 0.0)
    a2 = jnp.dot(hj_ref[...], r2, preferred_element_type=jnp.float32)
    out_ref[...] = jnp.dot(a2 * ew2, s2, preferred_element_type=jnp.float32)


def _combine1_body(p_ref, x_ref, rw_ref, bias_ref, out_ref):
    acc = p_ref[0] + p_ref[1]
    cnt = acc[:, 4:5]
    inv = 1.0 / jnp.maximum(cnt, 1.0)
    root = jnp.dot(x_ref[...], rw_ref[...],
                   preferred_element_type=jnp.float32) + bias_ref[...]
    h1 = jnp.maximum(acc * inv + root, 0.0)
    col = lax.broadcasted_iota(jnp.int32, acc.shape, 1)
    out_ref[...] = jnp.where(col == 4, inv, jnp.where(col < 4, h1, 0.0))


def _final_body(p_ref, h1_ref, rw_ref, b2_ref, c1w_ref, c1b_ref, c2w_ref,
                c2b_ref, out_ref):
    acc = p_ref[0] + p_ref[1]
    h1 = h1_ref[...]
    inv = h1[:, 4:5]
    root = jnp.dot(h1, rw_ref[...], preferred_element_type=jnp.float32)
    h2 = jnp.maximum(acc * inv + root + b2_ref[...], 0.0)
    h3 = jnp.maximum(
        jnp.dot(h2, c1w_ref[...], preferred_element_type=jnp.float32)
        + c1b_ref[...], 0.0)
    out_ref[...] = jnp.dot(h3, c2w_ref[...],
                           preferred_element_type=jnp.float32) + c2b_ref[...]


def _full_spec(shape):
    return pl.BlockSpec(shape, lambda i: tuple(0 for _ in shape))


def _msg1(ea, xj, w1, b1, w2r, b2r, e_pad, blk=2048):
    grid = (e_pad // blk,)
    return pl.pallas_call(
        _msg1_body,
        grid=grid,
        in_specs=[
            pl.BlockSpec((blk, 4), lambda i: (i, 0)),
            pl.BlockSpec((blk, 128), lambda i: (i, 0)),
            _full_spec((4, 512)),
            _full_spec((1, 512)),
            _full_spec((4, 512, 128)),
            _full_spec((4, 128)),
        ],
        out_specs=pl.BlockSpec((blk, 16), lambda i: (i, 0)),
        out_shape=jax.ShapeDtypeStruct((e_pad, 16), jnp.float32),
    )(ea, xj, w1, b1, w2r, b2r)


def _msg2(ea, hj, w1, b1, w2, b2, e_pad, blk=4096):
    grid = (e_pad // blk,)
    return pl.pallas_call(
        _msg2_body,
        grid=grid,
        in_specs=[
            pl.BlockSpec((blk, 4), lambda i: (i, 0)),
            pl.BlockSpec((blk, 16), lambda i: (i, 0)),
            _full_spec((4, 16)),
            _full_spec((1, 16)),
            _full_spec((16, 16)),
            _full_spec((1, 16)),
        ],
        out_specs=pl.BlockSpec((blk, 16), lambda i: (i, 0)),
        out_shape=jax.ShapeDtypeStruct((e_pad, 16), jnp.float32),
    )(ea, hj, w1, b1, w2, b2)


def _combine1(partial, x_pad, rw16, b16, n_pad, blk=1024):
    grid = (n_pad // blk,)
    return pl.pallas_call(
        _combine1_body,
        grid=grid,
        in_specs=[
            pl.BlockSpec((2, blk, 16), lambda i: (0, i, 0)),
            pl.BlockSpec((blk, 128), lambda i: (i, 0)),
            _full_spec((128, 16)),
            _full_spec((1, 16)),
        ],
        out_specs=pl.BlockSpec((blk, 16), lambda i: (i, 0)),
        out_shape=jax.ShapeDtypeStruct((n_pad, 16), jnp.float32),
    )(partial, x_pad, rw16, b16)


def _final(partial2, h1p, rw16, b16, c1w, c1b, c2w, c2b, n_pad, blk=1024):
    grid = (n_pad // blk,)
    return pl.pallas_call(
        _final_body,
        grid=grid,
        in_specs=[
            pl.BlockSpec((2, blk, 16), lambda i: (0, i, 0)),
            pl.BlockSpec((blk, 16), lambda i: (i, 0)),
            _full_spec((16, 16)),
            _full_spec((1, 16)),
            _full_spec((16, 16)),
            _full_spec((1, 16)),
            _full_spec((16, 40)),
            _full_spec((1, 40)),
        ],
        out_specs=pl.BlockSpec((blk, 40), lambda i: (i, 0)),
        out_shape=jax.ShapeDtypeStruct((n_pad, 40), jnp.float32),
    )(partial2, h1p, rw16, b16, c1w, c1b, c2w, c2b)


def kernel(x, edge_index, edge_attr, nn1_w1, nn1_b1, nn1_w2, nn1_b2, root1_w,
           bias1, nn2_w1, nn2_b1, nn2_w2, nn2_b2, root2_w, bias2, cls_w1,
           cls_b1, cls_w2, cls_b2):
    n, in_ch = x.shape
    e = edge_index.shape[1]
    gran = _NW * _CH
    e_pad = ((e + gran - 1) // gran) * gran
    n_pad = ((n + 16 + 1023) // 1024) * 1024

    src = jnp.concatenate([edge_index[0], jnp.zeros((e_pad - e,), jnp.int32)])
    dst = jnp.concatenate(
        [edge_index[1], jnp.full((e_pad - e,), n, jnp.int32)])
    ea = jnp.concatenate(
        [edge_attr, jnp.zeros((e_pad - e, edge_attr.shape[1]), jnp.float32)])
    zeros16 = jnp.zeros((n_pad, 16), jnp.float32)
    x_pad = jnp.concatenate([x, jnp.zeros((n_pad - n, in_ch), jnp.float32)])

    hid = root1_w.shape[1]
    d1 = in_ch * hid
    w2r = jnp.stack([nn1_w2[:, o::hid] for o in range(hid)])
    b2r = jnp.stack([nn1_b2[o::hid] for o in range(hid)])
    rw16 = jnp.pad(root1_w, ((0, 0), (0, 16 - hid)))
    b16 = jnp.pad(bias1, (0, 16 - hid))[None, :]
    r2w16 = jnp.pad(root2_w, ((0, 16 - hid), (0, 16 - hid)))
    b2_16 = jnp.pad(bias2, (0, 16 - hid))[None, :]
    c1w16 = jnp.pad(cls_w1, ((0, 16 - hid), (0, 16 - hid)))
    c1b16 = jnp.pad(cls_b1, (0, 16 - hid))[None, :]
    c2w16 = jnp.pad(cls_w2, ((0, 16 - hid), (0, 0)))
    c2b = cls_b2[None, :]

    xj = _sc_gather(x, src)
    msg1 = _msg1(ea, xj, nn1_w1, nn1_b1[None, :], w2r, b2r, e_pad)
    part1 = _sc_scatter_add(msg1, dst, zeros16, n_pad)
    h1p = _combine1(part1, x_pad, rw16, b16, n_pad)

    hj = _sc_gather(h1p, src)
    msg2 = _msg2(ea, hj, nn2_w1, nn2_b1[None, :], nn2_w2, nn2_b2[None, :],
                 e_pad)
    part2 = _sc_scatter_add(msg2, dst, zeros16, n_pad)
    out = _final(part2, h1p, r2w16, b2_16, c1w16, c1b16, c2w16, c2b, n_pad)
    return out[:n]


# speedup vs baseline: 2.9989x; 1.2001x over previous
import functools

import jax
import jax.numpy as jnp
from jax import lax
from jax.experimental import pallas as pl
from jax.experimental.pallas import tpu as pltpu
from jax.experimental.pallas import tpu_sc as plsc

_NW = 32
_CH = 128


def _sc_gather(table, idx2d, nbuf=4):
    n_rows = idx2d.shape[0]
    e_pad = n_rows * _CH
    d = table.shape[1]
    b_per_w = e_pad // _NW
    n_ch = b_per_w // _CH
    mesh = plsc.VectorSubcoreMesh(core_axis_name="c", subcore_axis_name="s")

    @functools.partial(
        pl.kernel, mesh=mesh,
        out_type=jax.ShapeDtypeStruct((e_pad, d), jnp.float32),
        compiler_params=pltpu.CompilerParams(use_tc_tiling_on_sc=False),
        scratch_types=[
            pltpu.VMEM((n_ch, _CH), jnp.int32),
        ] + [pltpu.VMEM((_CH, d), jnp.float32) for _ in range(nbuf)]
          + [pltpu.SemaphoreType.DMA for _ in range(nbuf)],
    )
    def k(table_hbm, idx_hbm, out_hbm, idx_v, *bufs):
        rows = bufs[:nbuf]
        sems = bufs[nbuf:]
        wid = lax.axis_index("s") * 2 + lax.axis_index("c")
        ebase = wid * b_per_w
        pltpu.sync_copy(idx_hbm.at[pl.ds(wid * n_ch, n_ch)], idx_v)
        for b in range(nbuf):
            pltpu.async_copy(table_hbm.at[idx_v.at[b]], rows[b], sems[b])

        def outer(c0, carry):
            for b in range(nbuf):
                c = c0 * nbuf + b
                pltpu.make_async_copy(
                    table_hbm.at[idx_v.at[b]], rows[b], sems[b]).wait()
                pltpu.sync_copy(rows[b],
                                out_hbm.at[pl.ds(ebase + c * _CH, _CH)])
                nxt = c + nbuf

                @pl.when(nxt < n_ch)
                def _():
                    pltpu.async_copy(
                        table_hbm.at[idx_v.at[nxt]], rows[b], sems[b])
            return carry

        lax.fori_loop(0, n_ch // nbuf, outer, 0)

    return k(table, idx2d)


def _sc_scatter_add(vals, idx2d, zeros, n_acc):
    n_rows = idx2d.shape[0]
    e_pad = n_rows * _CH
    b_per_w = e_pad // _NW
    n_ch = b_per_w // _CH
    n_slice = n_acc // 16
    mesh = plsc.VectorSubcoreMesh(core_axis_name="c", subcore_axis_name="s")

    @functools.partial(
        pl.kernel, mesh=mesh,
        out_type=jax.ShapeDtypeStruct((2, n_acc, 16), jnp.float32),
        compiler_params=pltpu.CompilerParams(use_tc_tiling_on_sc=False),
        scratch_types=[
            pltpu.VMEM((n_ch, _CH), jnp.int32),
            pltpu.VMEM((b_per_w, 16), jnp.float32),
            pltpu.VMEM_SHARED((n_acc, 16), jnp.float32),
            pltpu.SemaphoreType.DMA,
        ],
    )
    def k(vals_hbm, idx_hbm, zeros_hbm, out_hbm, idx_v, vals_v, acc_sh, sem):
        cid = lax.axis_index("c")
        sid = lax.axis_index("s")
        wid = sid * 2 + cid
        base = wid * b_per_w
        pltpu.async_copy(vals_hbm.at[pl.ds(base, b_per_w)], vals_v, sem)
        pltpu.sync_copy(idx_hbm.at[pl.ds(wid * n_ch, n_ch)], idx_v)
        pltpu.sync_copy(zeros_hbm.at[pl.ds(sid * n_slice, n_slice)],
                        acc_sh.at[pl.ds(sid * n_slice, n_slice)])
        pltpu.make_async_copy(
            vals_hbm.at[pl.ds(base, b_per_w)], vals_v, sem).wait()
        plsc.subcore_barrier()

        def body(c, carry):
            pltpu.sync_copy(vals_v.at[pl.ds(c * _CH, _CH)],
                            acc_sh.at[idx_v.at[c]], add=True)
            return carry

        lax.fori_loop(0, n_ch, body, 0)
        plsc.subcore_barrier()
        pltpu.sync_copy(acc_sh.at[pl.ds(sid * n_slice, n_slice)],
                        out_hbm.at[cid, pl.ds(sid * n_slice, n_slice)])

    return k(vals, idx2d, zeros)


def _msg1_body(ea_ref, xj_ref, w1_ref, b1_ref, w2r_ref, b2r_ref, out_ref):
    b = ea_ref.shape[0]
    h = jnp.maximum(
        jnp.dot(ea_ref[...], w1_ref[...], preferred_element_type=jnp.float32)
        + b1_ref[...], 0.0)
    xj = xj_ref[...]
    col = lax.broadcasted_iota(jnp.int32, (b, 16), 1)
    m = jnp.where(col == 4, 1.0, 0.0)
    for o in range(4):
        c = jnp.dot(h, w2r_ref[o], preferred_element_type=jnp.float32)
        c = c + b2r_ref[o:o + 1, :]
        rs = jnp.sum(xj * c, axis=1, keepdims=True)
        m = m + rs * jnp.where(col == o, 1.0, 0.0)
    out_ref[...] = m


def _msg2_body(ea_ref, hj_ref, w1_ref, b1_ref, w2_ref, b2_ref, out_ref):
    h2e = jnp.maximum(
        jnp.dot(ea_ref[...], w1_ref[...], preferred_element_type=jnp.float32)
        + b1_ref[...], 0.0)
    ew2 = jnp.dot(h2e, w2_ref[...], preferred_element_type=jnp.float32) \
        + b2_ref[...]
    ri = lax.broadcasted_iota(jnp.int32, (16, 16), 0)
    rj = lax.broadcasted_iota(jnp.int32, (16, 16), 1)
    r2 = jnp.where(ri == rj // 4, 1.0, 0.0)
    s2 = jnp.where(rj == ri % 4, 1.0, 0.0)
    a2 = jnp.dot(hj_ref[...], r2, preferred_element_type=jnp.float32)
    out_ref[...] = jnp.dot(a2 * ew2, s2, preferred_element_type=jnp.float32)


def _combine1_body(p_ref, x_ref, rw_ref, bias_ref, out_ref):
    acc = p_ref[0] + p_ref[1]
    cnt = acc[:, 4:5]
    inv = 1.0 / jnp.maximum(cnt, 1.0)
    root = jnp.dot(x_ref[...], rw_ref[...],
                   preferred_element_type=jnp.float32) + bias_ref[...]
    h1 = jnp.maximum(acc * inv + root, 0.0)
    col = lax.broadcasted_iota(jnp.int32, acc.shape, 1)
    out_ref[...] = jnp.where(col == 4, inv, jnp.where(col < 4, h1, 0.0))


def _final_body(p_ref, h1_ref, rw_ref, b2_ref, c1w_ref, c1b_ref, c2w_ref,
                c2b_ref, out_ref):
    acc = p_ref[0] + p_ref[1]
    h1 = h1_ref[...]
    inv = h1[:, 4:5]
    root = jnp.dot(h1, rw_ref[...], preferred_element_type=jnp.float32)
    h2 = jnp.maximum(acc * inv + root + b2_ref[...], 0.0)
    h3 = jnp.maximum(
        jnp.dot(h2, c1w_ref[...], preferred_element_type=jnp.float32)
        + c1b_ref[...], 0.0)
    out_ref[...] = jnp.dot(h3, c2w_ref[...],
                           preferred_element_type=jnp.float32) + c2b_ref[...]


def _full_spec(shape):
    return pl.BlockSpec(shape, lambda i: tuple(0 for _ in shape))


def _msg1(ea, xj, w1, b1, w2r, b2r, e_pad, blk=2048):
    grid = (e_pad // blk,)
    return pl.pallas_call(
        _msg1_body,
        grid=grid,
        in_specs=[
            pl.BlockSpec((blk, 4), lambda i: (i, 0)),
            pl.BlockSpec((blk, 128), lambda i: (i, 0)),
            _full_spec((4, 512)),
            _full_spec((1, 512)),
            _full_spec((4, 512, 128)),
            _full_spec((4, 128)),
        ],
        out_specs=pl.BlockSpec((blk, 16), lambda i: (i, 0)),
        out_shape=jax.ShapeDtypeStruct((e_pad, 16), jnp.float32),
    )(ea, xj, w1, b1, w2r, b2r)


def _msg2(ea, hj, w1, b1, w2, b2, e_pad, blk=4096):
    grid = (e_pad // blk,)
    return pl.pallas_call(
        _msg2_body,
        grid=grid,
        in_specs=[
            pl.BlockSpec((blk, 4), lambda i: (i, 0)),
            pl.BlockSpec((blk, 16), lambda i: (i, 0)),
            _full_spec((4, 16)),
            _full_spec((1, 16)),
            _full_spec((16, 16)),
            _full_spec((1, 16)),
        ],
        out_specs=pl.BlockSpec((blk, 16), lambda i: (i, 0)),
        out_shape=jax.ShapeDtypeStruct((e_pad, 16), jnp.float32),
    )(ea, hj, w1, b1, w2, b2)


def _combine1(partial, x_pad, rw16, b16, n_pad, blk=1024):
    grid = (n_pad // blk,)
    return pl.pallas_call(
        _combine1_body,
        grid=grid,
        in_specs=[
            pl.BlockSpec((2, blk, 16), lambda i: (0, i, 0)),
            pl.BlockSpec((blk, 128), lambda i: (i, 0)),
            _full_spec((128, 16)),
            _full_spec((1, 16)),
        ],
        out_specs=pl.BlockSpec((blk, 16), lambda i: (i, 0)),
        out_shape=jax.ShapeDtypeStruct((n_pad, 16), jnp.float32),
    )(partial, x_pad, rw16, b16)


def _final(partial2, h1p, rw16, b16, c1w, c1b, c2w, c2b, n_pad, blk=1024):
    grid = (n_pad // blk,)
    return pl.pallas_call(
        _final_body,
        grid=grid,
        in_specs=[
            pl.BlockSpec((2, blk, 16), lambda i: (0, i, 0)),
            pl.BlockSpec((blk, 16), lambda i: (i, 0)),
            _full_spec((16, 16)),
            _full_spec((1, 16)),
            _full_spec((16, 16)),
            _full_spec((1, 16)),
            _full_spec((16, 40)),
            _full_spec((1, 40)),
        ],
        out_specs=pl.BlockSpec((blk, 40), lambda i: (i, 0)),
        out_shape=jax.ShapeDtypeStruct((n_pad, 40), jnp.float32),
    )(partial2, h1p, rw16, b16, c1w, c1b, c2w, c2b)


def kernel(x, edge_index, edge_attr, nn1_w1, nn1_b1, nn1_w2, nn1_b2, root1_w,
           bias1, nn2_w1, nn2_b1, nn2_w2, nn2_b2, root2_w, bias2, cls_w1,
           cls_b1, cls_w2, cls_b2):
    n, in_ch = x.shape
    e = edge_index.shape[1]
    gran = _NW * _CH
    e_pad = ((e + gran - 1) // gran) * gran
    n_pad = ((n + 16 + 1023) // 1024) * 1024

    src = jnp.concatenate(
        [edge_index[0], jnp.zeros((e_pad - e,), jnp.int32)]
    ).reshape(e_pad // _CH, _CH)
    dst = jnp.concatenate(
        [edge_index[1], jnp.full((e_pad - e,), n, jnp.int32)]
    ).reshape(e_pad // _CH, _CH)
    ea = jnp.concatenate(
        [edge_attr, jnp.zeros((e_pad - e, edge_attr.shape[1]), jnp.float32)])
    zeros16 = jnp.zeros((n_pad, 16), jnp.float32)
    x_pad = jnp.concatenate([x, jnp.zeros((n_pad - n, in_ch), jnp.float32)])

    hid = root1_w.shape[1]
    d1 = in_ch * hid
    w2r = jnp.stack([nn1_w2[:, o::hid] for o in range(hid)])
    b2r = jnp.stack([nn1_b2[o::hid] for o in range(hid)])
    rw16 = jnp.pad(root1_w, ((0, 0), (0, 16 - hid)))
    b16 = jnp.pad(bias1, (0, 16 - hid))[None, :]
    r2w16 = jnp.pad(root2_w, ((0, 16 - hid), (0, 16 - hid)))
    b2_16 = jnp.pad(bias2, (0, 16 - hid))[None, :]
    c1w16 = jnp.pad(cls_w1, ((0, 16 - hid), (0, 16 - hid)))
    c1b16 = jnp.pad(cls_b1, (0, 16 - hid))[None, :]
    c2w16 = jnp.pad(cls_w2, ((0, 16 - hid), (0, 0)))
    c2b = cls_b2[None, :]

    xj = _sc_gather(x, src)
    msg1 = _msg1(ea, xj, nn1_w1, nn1_b1[None, :], w2r, b2r, e_pad)
    part1 = _sc_scatter_add(msg1, dst, zeros16, n_pad)
    h1p = _combine1(part1, x_pad, rw16, b16, n_pad)

    hj = _sc_gather(h1p, src)
    msg2 = _msg2(ea, hj, nn2_w1, nn2_b1[None, :], nn2_w2, nn2_b2[None, :],
                 e_pad)
    part2 = _sc_scatter_add(msg2, dst, zeros16, n_pad)
    out = _final(part2, h1p, r2w16, b2_16, c1w16, c1b16, c2w16, c2b, n_pad)
    return out[:n]
